# Initial kernel scaffold; baseline (speedup 1.0000x reference)
import jax
import jax.numpy as jnp
from jax.experimental import pallas as pl


def kernel(x0, x1, edge_index, gcn0_W0, gcn0_b0, gcn0_W1, gcn0_b1, gcn1_W0, gcn1_b0, gcn1_W1, gcn1_b1, dec_W, dec_b, wsum):
    raise NotImplementedError("write your pallas kernel here")


# speedup vs baseline: 2.4190x; 2.4190x over previous
import functools

import jax
import jax.numpy as jnp
from jax import lax
from jax.experimental import pallas as pl
from jax.experimental.pallas import tpu as pltpu
from jax.experimental.pallas import tpu_sc as plsc

N = 10000
H = 128
E = 160000
NSC = 16
CHUNK = 128
IB = 8
C = 80
CB = C // IB
E_PAD = NSC * C * CHUNK
NACC = 10112
RPT = NACC // NSC

f32 = jnp.float32


@functools.cache
def _build_segsum_sc():
    mesh = plsc.VectorSubcoreMesh(core_axis_name="c", subcore_axis_name="s")

    @functools.partial(
        pl.kernel,
        out_type=jax.ShapeDtypeStruct((2 * NACC, H), f32),
        mesh=mesh,
        scratch_types=[
            pltpu.VMEM_SHARED((NACC, H), f32),
            pltpu.VMEM((IB, CHUNK), jnp.int32),
            pltpu.VMEM((IB, CHUNK), jnp.int32),
            pltpu.VMEM((CHUNK, H), f32),
            pltpu.SemaphoreType.DMA,
        ],
    )
    def _segsum_sc(hw_flat, src_idx, dst_idx, zrows,
                   out, acc, src_v, dst_v, gbuf, sem):
        c = lax.axis_index("c")
        s = lax.axis_index("s")
        w = c * NSC + s
        base = s * RPT
        pltpu.sync_copy(zrows.at[pl.ds(base, RPT)], acc.at[pl.ds(base, RPT)])
        plsc.subcore_barrier()

        def blk(b, carry):
            pltpu.sync_copy(src_idx.at[pl.ds(w * C + b * IB, IB)], src_v)
            pltpu.sync_copy(dst_idx.at[pl.ds(s * C + b * IB, IB)], dst_v)
            for j in range(IB):
                pltpu.async_copy(hw_flat.at[src_v.at[j]], gbuf, sem).wait()
                pltpu.sync_copy(gbuf, acc.at[dst_v.at[j]], add=True)
            return carry

        lax.fori_loop(0, CB, blk, 0)
        plsc.subcore_barrier()
        obase = c * NACC + s * RPT
        pltpu.sync_copy(acc.at[pl.ds(s * RPT, RPT)], out.at[pl.ds(obase, RPT)])

    return _segsum_sc


@functools.cache
def _build_deg_sc():
    mesh = plsc.VectorSubcoreMesh(core_axis_name="c", subcore_axis_name="s")

    @functools.partial(
        pl.kernel,
        out_type=jax.ShapeDtypeStruct((2 * NACC, H), f32),
        mesh=mesh,
        scratch_types=[
            pltpu.VMEM_SHARED((NACC, H), f32),
            pltpu.VMEM((IB, CHUNK), jnp.int32),
            pltpu.VMEM((CHUNK, H), f32),
        ],
    )
    def _deg_sc(dst_idx, zrows, ones128, out, acc, dst_v, ones_v):
        c = lax.axis_index("c")
        s = lax.axis_index("s")
        base = s * RPT
        pltpu.sync_copy(ones128, ones_v)
        pltpu.sync_copy(zrows.at[pl.ds(base, RPT)], acc.at[pl.ds(base, RPT)])
        plsc.subcore_barrier()

        def blk(b, carry):
            pltpu.sync_copy(dst_idx.at[pl.ds(s * C + b * IB, IB)], dst_v)
            for j in range(IB):
                pltpu.sync_copy(ones_v, acc.at[dst_v.at[j]], add=True)
            return carry

        lax.fori_loop(0, CB, blk, 0)
        plsc.subcore_barrier()
        obase = c * NACC + s * RPT
        pltpu.sync_copy(acc.at[pl.ds(s * RPT, RPT)], out.at[pl.ds(obase, RPT)])

    return _deg_sc


def _mm0(x_both, w_both, bn=1000):
    def body(x_ref, w_ref, o_ref):
        o_ref[0] = jnp.dot(x_ref[0], w_ref[0], preferred_element_type=f32)
    return pl.pallas_call(
        body,
        grid=(2, N // bn),
        in_specs=[
            pl.BlockSpec((1, bn, H), lambda g, i: (g, i, 0)),
            pl.BlockSpec((1, H, H), lambda g, i: (g, 0, 0)),
        ],
        out_specs=pl.BlockSpec((1, bn, H), lambda g, i: (g, i, 0)),
        out_shape=jax.ShapeDtypeStruct((2, N, H), f32),
    )(x_both, w_both)


def _mm1_fused(agg, deg16, b_both, w_both, bn=1000):
    def body(a_ref, d_ref, b_ref, w_ref, o_ref):
        d = jnp.maximum(d_ref[:, 0:1], 1.0)
        h = jnp.maximum(a_ref[0] / d + b_ref[0], 0.0)
        o_ref[0] = jnp.dot(h, w_ref[0], preferred_element_type=f32)
    return pl.pallas_call(
        body,
        grid=(2, N // bn),
        in_specs=[
            pl.BlockSpec((1, bn, H), lambda g, i: (g, i, 0)),
            pl.BlockSpec((bn, 16), lambda g, i: (i, 0)),
            pl.BlockSpec((1, 1, H), lambda g, i: (g, 0, 0)),
            pl.BlockSpec((1, H, H), lambda g, i: (g, 0, 0)),
        ],
        out_specs=pl.BlockSpec((1, bn, H), lambda g, i: (g, i, 0)),
        out_shape=jax.ShapeDtypeStruct((2, N, H), f32),
    )(agg, deg16, b_both, w_both)


def _final_fused(agg, deg16, b_both, ws_pad, dec_w, dec_b2, bn=1000):
    dw = dec_w.shape[1]

    def body(a_ref, d_ref, b_ref, ws_ref, w_ref, db_ref, z_ref, x_ref):
        d = jnp.maximum(d_ref[:, 0:1], 1.0)
        z0 = jnp.maximum(a_ref[0] / d + b_ref[0], 0.0)
        z1 = jnp.maximum(a_ref[1] / d + b_ref[1], 0.0)
        z = ws_ref[0, 0] * z0 + ws_ref[0, 1] * z1
        z_ref[...] = z
        x = jnp.dot(z, w_ref[...], preferred_element_type=f32) + db_ref[...]
        x_ref[...] = jnp.maximum(x, 0.0)

    return pl.pallas_call(
        body,
        grid=(N // bn,),
        in_specs=[
            pl.BlockSpec((2, bn, H), lambda i: (0, i, 0)),
            pl.BlockSpec((bn, 16), lambda i: (i, 0)),
            pl.BlockSpec((2, 1, H), lambda i: (0, 0, 0)),
            pl.BlockSpec((8, 128), lambda i: (0, 0)),
            pl.BlockSpec((H, dw), lambda i: (0, 0)),
            pl.BlockSpec((1, dw), lambda i: (0, 0)),
        ],
        out_specs=[
            pl.BlockSpec((bn, H), lambda i: (i, 0)),
            pl.BlockSpec((bn, dw), lambda i: (i, 0)),
        ],
        out_shape=[
            jax.ShapeDtypeStruct((N, H), f32),
            jax.ShapeDtypeStruct((N, dw), f32),
        ],
    )(agg, deg16, b_both, ws_pad, dec_w, dec_b2)


def _zzt(z, bi=400):
    def body(zi_ref, zj_ref, o_ref):
        o_ref[...] = lax.dot_general(
            zi_ref[...], zj_ref[...], (((1,), (1,)), ((), ())),
            preferred_element_type=f32)
    return pl.pallas_call(
        body,
        grid=(N // bi,),
        in_specs=[
            pl.BlockSpec((bi, H), lambda i: (i, 0)),
            pl.BlockSpec((N, H), lambda i: (0, 0)),
        ],
        out_specs=pl.BlockSpec((bi, N), lambda i: (i, 0)),
        out_shape=jax.ShapeDtypeStruct((N, N), f32),
    )(z, z)


def _segsum(hw_both, src_idx, dst_idx, zrows):
    out = _build_segsum_sc()(hw_both.reshape(2 * N, H), src_idx, dst_idx,
                             zrows)
    return out.reshape(2, NACC, H)[:, :N, :]


def kernel(x0, x1, edge_index, gcn0_W0, gcn0_b0, gcn0_W1, gcn0_b1,
           gcn1_W0, gcn1_b0, gcn1_W1, gcn1_b1, dec_W, dec_b, wsum):
    src = edge_index[0].astype(jnp.int32)
    dst = edge_index[1].astype(jnp.int32)
    pad = E_PAD - E
    src_p = jnp.concatenate([src, jnp.zeros((pad,), jnp.int32)])
    dst_p = jnp.concatenate([dst, jnp.full((pad,), N, jnp.int32)])
    src_tiles = src_p.reshape(NSC * C, CHUNK)
    src_idx = jnp.concatenate([src_tiles, src_tiles + N])
    dst_idx = dst_p.reshape(NSC * C, CHUNK)

    zrows = jnp.zeros((NACC, H), f32)
    ones128 = jnp.ones((CHUNK, H), f32)

    x_both = jnp.stack([x0, x1])
    w0_both = jnp.stack([gcn0_W0, gcn1_W0])
    b0_both = jnp.stack([gcn0_b0, gcn1_b0]).reshape(2, 1, H)
    w1_both = jnp.stack([gcn0_W1, gcn1_W1])
    b1_both = jnp.stack([gcn0_b1, gcn1_b1]).reshape(2, 1, H)

    wtot = wsum.sum()
    ws_pad = jnp.zeros((8, 128), f32)
    ws_pad = ws_pad.at[0, 0].set(wsum[0, 0, 0] / wtot)
    ws_pad = ws_pad.at[0, 1].set(wsum[1, 0, 0] / wtot)
    dec_b2 = dec_b.reshape(1, -1)

    deg_out = _build_deg_sc()(dst_idx, zrows, ones128)
    deg16 = deg_out[:N, :16]
    hw0 = _mm0(x_both, w0_both)
    agg0 = _segsum(hw0, src_idx, dst_idx, zrows)
    hw1 = _mm1_fused(agg0, deg16, b0_both, w1_both)
    agg1 = _segsum(hw1, src_idx, dst_idx, zrows)
    Z, X_rec = _final_fused(agg1, deg16, b1_both, ws_pad, dec_W, dec_b2)
    A_rec = _zzt(Z)
    return (A_rec, X_rec)


# speedup vs baseline: 2.8299x; 1.1698x over previous
import functools

import jax
import jax.numpy as jnp
from jax import lax
from jax.experimental import pallas as pl
from jax.experimental.pallas import tpu as pltpu
from jax.experimental.pallas import tpu_sc as plsc

N = 10000
H = 128
E = 160000
NSC = 16
CHUNK = 128
IB = 8
C = 80
NB = C // IB
E_PAD = NSC * C * CHUNK
NACC = 10112
RPT = NACC // NSC

f32 = jnp.float32


NBUF = 2
NIS = 3


@functools.cache
def _build_segsum_sc():
    mesh = plsc.VectorSubcoreMesh(core_axis_name="c", subcore_axis_name="s")

    @functools.partial(
        pl.kernel,
        out_type=jax.ShapeDtypeStruct((2 * NACC, H), f32),
        mesh=mesh,
        scratch_types=[
            pltpu.VMEM_SHARED((NACC, H), f32),
            pltpu.VMEM((NIS * IB, CHUNK), jnp.int32),
            pltpu.VMEM((NIS * IB, CHUNK), jnp.int32),
            pltpu.VMEM((NBUF, CHUNK, H), f32),
            pltpu.SemaphoreType.DMA,
            pltpu.SemaphoreType.DMA,
            pltpu.SemaphoreType.DMA,
            pltpu.SemaphoreType.DMA,
            pltpu.SemaphoreType.DMA,
        ],
    )
    def _segsum_sc(hw_flat, src_idx, dst_idx, zrows,
                   out, acc, src_v, dst_v, gbuf, g0, g1, i0, i1, i2):
        gsems = (g0, g1)
        isems = (i0, i1, i2)
        c = lax.axis_index("c")
        s = lax.axis_index("s")
        w = c * NSC + s
        base = s * RPT
        pltpu.sync_copy(zrows.at[pl.ds(base, RPT)], acc.at[pl.ds(base, RPT)])
        pltpu.sync_copy(src_idx.at[pl.ds(w * C, IB)], src_v.at[pl.ds(0, IB)])
        pltpu.sync_copy(dst_idx.at[pl.ds(s * C, IB)], dst_v.at[pl.ds(0, IB)])
        ihs = [None] * NB
        if NB > 1:
            ihs[1] = (
                pltpu.async_copy(src_idx.at[pl.ds(w * C + IB, IB)],
                                 src_v.at[pl.ds(IB, IB)], isems[1]),
                pltpu.async_copy(dst_idx.at[pl.ds(s * C + IB, IB)],
                                 dst_v.at[pl.ds(IB, IB)], isems[1]),
            )
        plsc.subcore_barrier()

        hs = [None] * C
        for j in range(NBUF):
            hs[j] = pltpu.async_copy(hw_flat.at[src_v.at[j]], gbuf.at[j],
                                     gsems[j])
        for j in range(C):
            k = j % NBUF
            hs[j].wait()
            drow = ((j // IB) % NIS) * IB + j % IB
            pltpu.sync_copy(gbuf.at[k], acc.at[dst_v.at[drow]], add=True)
            nj = j + NBUF
            if nj < C:
                if nj % IB == 0:
                    bn = nj // IB
                    for h in ihs[bn]:
                        h.wait()
                    if bn + 1 < NB:
                        sl = ((bn + 1) % NIS) * IB
                        ihs[bn + 1] = (
                            pltpu.async_copy(
                                src_idx.at[pl.ds(w * C + (bn + 1) * IB, IB)],
                                src_v.at[pl.ds(sl, IB)],
                                isems[(bn + 1) % NIS]),
                            pltpu.async_copy(
                                dst_idx.at[pl.ds(s * C + (bn + 1) * IB, IB)],
                                dst_v.at[pl.ds(sl, IB)],
                                isems[(bn + 1) % NIS]),
                        )
                srow = ((nj // IB) % NIS) * IB + nj % IB
                hs[nj] = pltpu.async_copy(hw_flat.at[src_v.at[srow]],
                                          gbuf.at[k], gsems[k])

        plsc.subcore_barrier()
        obase = c * NACC + s * RPT
        pltpu.sync_copy(acc.at[pl.ds(s * RPT, RPT)], out.at[pl.ds(obase, RPT)])

    return _segsum_sc


@functools.cache
def _build_deg_sc():
    mesh = plsc.VectorSubcoreMesh(core_axis_name="c", subcore_axis_name="s")

    @functools.partial(
        pl.kernel,
        out_type=jax.ShapeDtypeStruct((2 * NACC, H), f32),
        mesh=mesh,
        scratch_types=[
            pltpu.VMEM_SHARED((NACC, H), f32),
            pltpu.VMEM((IB, CHUNK), jnp.int32),
            pltpu.VMEM((CHUNK, H), f32),
        ],
    )
    def _deg_sc(dst_idx, zrows, ones128, out, acc, dst_v, ones_v):
        c = lax.axis_index("c")
        s = lax.axis_index("s")
        base = s * RPT
        pltpu.sync_copy(ones128, ones_v)
        pltpu.sync_copy(zrows.at[pl.ds(base, RPT)], acc.at[pl.ds(base, RPT)])
        plsc.subcore_barrier()

        def blk(b, carry):
            pltpu.sync_copy(dst_idx.at[pl.ds(s * C + b * IB, IB)], dst_v)
            for j in range(IB):
                pltpu.sync_copy(ones_v, acc.at[dst_v.at[j]], add=True)
            return carry

        lax.fori_loop(0, NB, blk, 0)
        plsc.subcore_barrier()
        obase = c * NACC + s * RPT
        pltpu.sync_copy(acc.at[pl.ds(s * RPT, RPT)], out.at[pl.ds(obase, RPT)])

    return _deg_sc


def _mm0(x_both, w_both, bn=1000):
    def body(x_ref, w_ref, o_ref):
        o_ref[0] = jnp.dot(x_ref[0], w_ref[0], preferred_element_type=f32)
    return pl.pallas_call(
        body,
        grid=(2, N // bn),
        in_specs=[
            pl.BlockSpec((1, bn, H), lambda g, i: (g, i, 0)),
            pl.BlockSpec((1, H, H), lambda g, i: (g, 0, 0)),
        ],
        out_specs=pl.BlockSpec((1, bn, H), lambda g, i: (g, i, 0)),
        out_shape=jax.ShapeDtypeStruct((2, N, H), f32),
    )(x_both, w_both)


def _mm1_fused(agg, deg16, b_both, w_both, bn=1000):
    def body(a_ref, d_ref, b_ref, w_ref, o_ref):
        d = jnp.maximum(d_ref[:, 0:1], 1.0)
        h = jnp.maximum(a_ref[0] / d + b_ref[0], 0.0)
        o_ref[0] = jnp.dot(h, w_ref[0], preferred_element_type=f32)
    return pl.pallas_call(
        body,
        grid=(2, N // bn),
        in_specs=[
            pl.BlockSpec((1, bn, H), lambda g, i: (g, i, 0)),
            pl.BlockSpec((bn, 16), lambda g, i: (i, 0)),
            pl.BlockSpec((1, 1, H), lambda g, i: (g, 0, 0)),
            pl.BlockSpec((1, H, H), lambda g, i: (g, 0, 0)),
        ],
        out_specs=pl.BlockSpec((1, bn, H), lambda g, i: (g, i, 0)),
        out_shape=jax.ShapeDtypeStruct((2, N, H), f32),
    )(agg, deg16, b_both, w_both)


def _final_fused(agg, deg16, b_both, ws_pad, dec_w, dec_b2, bn=1000):
    dw = dec_w.shape[1]

    def body(a_ref, d_ref, b_ref, ws_ref, w_ref, db_ref, z_ref, x_ref):
        d = jnp.maximum(d_ref[:, 0:1], 1.0)
        z0 = jnp.maximum(a_ref[0] / d + b_ref[0], 0.0)
        z1 = jnp.maximum(a_ref[1] / d + b_ref[1], 0.0)
        z = ws_ref[0, 0] * z0 + ws_ref[0, 1] * z1
        z_ref[...] = z
        x = jnp.dot(z, w_ref[...], preferred_element_type=f32) + db_ref[...]
        x_ref[...] = jnp.maximum(x, 0.0)

    return pl.pallas_call(
        body,
        grid=(N // bn,),
        in_specs=[
            pl.BlockSpec((2, bn, H), lambda i: (0, i, 0)),
            pl.BlockSpec((bn, 16), lambda i: (i, 0)),
            pl.BlockSpec((2, 1, H), lambda i: (0, 0, 0)),
            pl.BlockSpec((8, 128), lambda i: (0, 0)),
            pl.BlockSpec((H, dw), lambda i: (0, 0)),
            pl.BlockSpec((1, dw), lambda i: (0, 0)),
        ],
        out_specs=[
            pl.BlockSpec((bn, H), lambda i: (i, 0)),
            pl.BlockSpec((bn, dw), lambda i: (i, 0)),
        ],
        out_shape=[
            jax.ShapeDtypeStruct((N, H), f32),
            jax.ShapeDtypeStruct((N, dw), f32),
        ],
    )(agg, deg16, b_both, ws_pad, dec_w, dec_b2)


def _zzt(z, bi=400):
    def body(zi_ref, zj_ref, o_ref):
        o_ref[...] = lax.dot_general(
            zi_ref[...], zj_ref[...], (((1,), (1,)), ((), ())),
            preferred_element_type=f32)
    return pl.pallas_call(
        body,
        grid=(N // bi,),
        in_specs=[
            pl.BlockSpec((bi, H), lambda i: (i, 0)),
            pl.BlockSpec((N, H), lambda i: (0, 0)),
        ],
        out_specs=pl.BlockSpec((bi, N), lambda i: (i, 0)),
        out_shape=jax.ShapeDtypeStruct((N, N), f32),
    )(z, z)


def _segsum(hw_both, src_idx, dst_idx, zrows):
    out = _build_segsum_sc()(hw_both.reshape(2 * N, H), src_idx, dst_idx,
                             zrows)
    return out.reshape(2, NACC, H)[:, :N, :]


def kernel(x0, x1, edge_index, gcn0_W0, gcn0_b0, gcn0_W1, gcn0_b1,
           gcn1_W0, gcn1_b0, gcn1_W1, gcn1_b1, dec_W, dec_b, wsum):
    src = edge_index[0].astype(jnp.int32)
    dst = edge_index[1].astype(jnp.int32)
    pad = E_PAD - E
    src_p = jnp.concatenate([src, jnp.zeros((pad,), jnp.int32)])
    dst_p = jnp.concatenate([dst, jnp.full((pad,), N, jnp.int32)])
    src_tiles = src_p.reshape(NSC * C, CHUNK)
    src_idx = jnp.concatenate([src_tiles, src_tiles + N])
    dst_idx = dst_p.reshape(NSC * C, CHUNK)

    zrows = jnp.zeros((NACC, H), f32)
    ones128 = jnp.ones((CHUNK, H), f32)

    x_both = jnp.stack([x0, x1])
    w0_both = jnp.stack([gcn0_W0, gcn1_W0])
    b0_both = jnp.stack([gcn0_b0, gcn1_b0]).reshape(2, 1, H)
    w1_both = jnp.stack([gcn0_W1, gcn1_W1])
    b1_both = jnp.stack([gcn0_b1, gcn1_b1]).reshape(2, 1, H)

    wtot = wsum.sum()
    ws_pad = jnp.zeros((8, 128), f32)
    ws_pad = ws_pad.at[0, 0].set(wsum[0, 0, 0] / wtot)
    ws_pad = ws_pad.at[0, 1].set(wsum[1, 0, 0] / wtot)
    dec_b2 = dec_b.reshape(1, -1)

    deg16 = _build_deg_sc()(dst_idx, zrows, ones128)[:N, :16]
    hw0 = _mm0(x_both, w0_both)
    agg0 = _segsum(hw0, src_idx, dst_idx, zrows)
    hw1 = _mm1_fused(agg0, deg16, b0_both, w1_both)
    agg1 = _segsum(hw1, src_idx, dst_idx, zrows)
    Z, X_rec = _final_fused(agg1, deg16, b1_both, ws_pad, dec_W, dec_b2)
    A_rec = _zzt(Z)
    return (A_rec, X_rec)
